# Initial kernel scaffold; baseline (speedup 1.0000x reference)
import jax
import jax.numpy as jnp
from jax.experimental import pallas as pl


def kernel(im1, im2, C, M1, M2):
    raise NotImplementedError("write your pallas kernel here")


# speedup vs baseline: 1.4904x; 1.4904x over previous
import functools

import jax
import jax.numpy as jnp
from jax import lax
from jax.experimental import pallas as pl
from jax.experimental.pallas import tpu as pltpu
from jax.experimental.pallas import tpu_sc as plsc

D = 224
HW = D * D
N = 64
CH = 3
NW = 32
SAMPLES_PER_W = N // NW
CHUNK_ROWS = 16
CHUNK = CHUNK_ROWS * D
NCHUNKS = HW // CHUNK
VECS_PER_ROW = D // 16


def _body(im1_hbm, im2_hbm, c_hbm, m1_hbm, m2_hbm, out_hbm,
          g1v, g2v, c0v, c1v, m1v, m2v, ov):
    cid = lax.axis_index("c")
    sid = lax.axis_index("s")
    wid = sid * 2 + cid
    tbase = lax.iota(jnp.int32, 16) * D

    for t in range(SAMPLES_PER_W):
        n = wid * SAMPLES_PER_W + t
        for ch in range(CH):
            imbase = (n * CH + ch) * HW
            cbase = n * 2 * HW
            pltpu.sync_copy(im1_hbm.at[pl.ds(imbase, HW)], g1v)
            pltpu.sync_copy(im2_hbm.at[pl.ds(imbase, HW)], g2v)

            def chunk_body(cki, _, imbase=imbase, cbase=cbase):
                k0 = cki * CHUNK
                pltpu.sync_copy(c_hbm.at[pl.ds(cbase + k0, CHUNK)], c0v)
                pltpu.sync_copy(c_hbm.at[pl.ds(cbase + HW + k0, CHUNK)], c1v)
                pltpu.sync_copy(m1_hbm.at[pl.ds(imbase + k0, CHUNK)], m1v)
                pltpu.sync_copy(m2_hbm.at[pl.ds(imbase + k0, CHUNK)], m2v)
                x0 = cki * CHUNK_ROWS

                def row_body(xr, _):
                    x = x0 + xr
                    base = xr * D
                    for j in range(VECS_PER_ROW):
                        off = base + j * 16
                        c0 = c0v[pl.ds(off, 16)]
                        c1 = c1v[pl.ds(off, 16)]
                        dd = c0 + c1 * D
                        tv = tbase + (j * 16 * D + x)
                        i1 = jnp.minimum(tv + dd, HW - 1)
                        i2 = jnp.maximum(tv - dd, 0)
                        g1 = plsc.load_gather(g1v, [i1])
                        g2 = plsc.load_gather(g2v, [i2])
                        ov[pl.ds(off, 16)] = (g1 * m1v[pl.ds(off, 16)]
                                              + g2 * m2v[pl.ds(off, 16)])
                    return 0

                lax.fori_loop(0, CHUNK_ROWS, row_body, 0)
                pltpu.sync_copy(ov, out_hbm.at[pl.ds(imbase + k0, CHUNK)])
                return 0

            lax.fori_loop(0, NCHUNKS, chunk_body, 0)


@jax.jit
def _run(im1f, im2f, cf, m1f, m2f):
    mesh = plsc.VectorSubcoreMesh(core_axis_name="c", subcore_axis_name="s")
    kern = functools.partial(
        pl.kernel,
        mesh=mesh,
        compiler_params=pltpu.CompilerParams(needs_layout_passes=False),
        out_type=jax.ShapeDtypeStruct((N * CH * HW,), jnp.float32),
        scratch_types=[
            pltpu.VMEM((HW,), jnp.float32),
            pltpu.VMEM((HW,), jnp.float32),
            pltpu.VMEM((CHUNK,), jnp.int32),
            pltpu.VMEM((CHUNK,), jnp.int32),
            pltpu.VMEM((CHUNK,), jnp.float32),
            pltpu.VMEM((CHUNK,), jnp.float32),
            pltpu.VMEM((CHUNK,), jnp.float32),
        ],
    )(_body)
    return kern(im1f, im2f, cf, m1f, m2f)


def kernel(im1, im2, C, M1, M2):
    outf = _run(
        im1.reshape(N * CH * HW),
        im2.reshape(N * CH * HW),
        C.reshape(N * 2 * HW),
        M1.reshape(N * CH * HW),
        M2.reshape(N * CH * HW),
    )
    return outf.reshape(N, CH, D, D)


# speedup vs baseline: 2.0542x; 1.3783x over previous
import functools

import jax
import jax.numpy as jnp
from jax import lax
from jax.experimental import pallas as pl
from jax.experimental.pallas import tpu as pltpu
from jax.experimental.pallas import tpu_sc as plsc

D = 224
HW = D * D
N = 64
CH = 3
NW = 32
SAMPLES_PER_W = N // NW
NTASK = SAMPLES_PER_W * CH
CHUNK_ROWS = 8
CHUNK = CHUNK_ROWS * D
NCHUNKS = HW // CHUNK
VECS_PER_ROW = D // 16


def _body(im1_hbm, im2_hbm, c_hbm, m1_hbm, m2_hbm, out_hbm,
          g1v, g2v, c0v, c1v, m1v, m2v, ov, sin0, sin1, sout0, sout1):
    cid = lax.axis_index("c")
    sid = lax.axis_index("s")
    wid = sid * 2 + cid
    tbase = lax.iota(jnp.int32, 16) * D
    sin = (sin0, sin1)
    sout = (sout0, sout1)

    def in_copies(slot, cbase, imbase, k0):
        return (
            pltpu.make_async_copy(c_hbm.at[pl.ds(cbase + k0, CHUNK)],
                                  c0v.at[slot], sin[slot]),
            pltpu.make_async_copy(c_hbm.at[pl.ds(cbase + HW + k0, CHUNK)],
                                  c1v.at[slot], sin[slot]),
            pltpu.make_async_copy(m1_hbm.at[pl.ds(imbase + k0, CHUNK)],
                                  m1v.at[slot], sin[slot]),
            pltpu.make_async_copy(m2_hbm.at[pl.ds(imbase + k0, CHUNK)],
                                  m2v.at[slot], sin[slot]),
        )

    def out_copy(slot, imbase, k0):
        return pltpu.make_async_copy(ov.at[slot],
                                     out_hbm.at[pl.ds(imbase + k0, CHUNK)],
                                     sout[slot])

    def task_body(t, _):
        n = wid * SAMPLES_PER_W + t // CH
        ch = t % CH
        imbase = (n * CH + ch) * HW
        cbase = n * 2 * HW
        pltpu.sync_copy(im1_hbm.at[pl.ds(imbase, HW)], g1v)
        pltpu.sync_copy(im2_hbm.at[pl.ds(imbase, HW)], g2v)

        for c in in_copies(0, cbase, imbase, 0):
            c.start()

        def compute_chunk(g, slot):
            x0 = g * CHUNK_ROWS

            def row_body(xr, _):
                x = x0 + xr
                base = xr * D
                for j in range(VECS_PER_ROW):
                    off = base + j * 16
                    c0 = c0v[slot, pl.ds(off, 16)]
                    c1 = c1v[slot, pl.ds(off, 16)]
                    dd = c0 + c1 * D
                    tv = tbase + (j * 16 * D + x)
                    i1 = jnp.minimum(tv + dd, HW - 1)
                    i2 = jnp.maximum(tv - dd, 0)
                    g1 = plsc.load_gather(g1v, [i1])
                    g2 = plsc.load_gather(g2v, [i2])
                    ov[slot, pl.ds(off, 16)] = (g1 * m1v[slot, pl.ds(off, 16)]
                                                + g2 * m2v[slot, pl.ds(off, 16)])
                return 0

            lax.fori_loop(0, CHUNK_ROWS, row_body, 0)

        def pair_body(i, _):
            for sub in range(2):
                g = i * 2 + sub
                slot = sub
                nxt = g + 1

                @pl.when(nxt < NCHUNKS)
                def _():
                    for c in in_copies(1 - slot, cbase, imbase, nxt * CHUNK):
                        c.start()

                for c in in_copies(slot, cbase, imbase, g * CHUNK):
                    c.wait()

                @pl.when(g >= 2)
                def _():
                    out_copy(slot, imbase, g * CHUNK).wait()

                compute_chunk(g, slot)
                out_copy(slot, imbase, g * CHUNK).start()
            return 0

        lax.fori_loop(0, NCHUNKS // 2, pair_body, 0)
        out_copy(0, imbase, (NCHUNKS - 2) * CHUNK).wait()
        out_copy(1, imbase, (NCHUNKS - 1) * CHUNK).wait()
        return 0

    lax.fori_loop(0, NTASK, task_body, 0)


@jax.jit
def _run(im1f, im2f, cf, m1f, m2f):
    mesh = plsc.VectorSubcoreMesh(core_axis_name="c", subcore_axis_name="s")
    kern = functools.partial(
        pl.kernel,
        mesh=mesh,
        compiler_params=pltpu.CompilerParams(needs_layout_passes=False),
        out_type=jax.ShapeDtypeStruct((N * CH * HW,), jnp.float32),
        scratch_types=[
            pltpu.VMEM((HW,), jnp.float32),
            pltpu.VMEM((HW,), jnp.float32),
            pltpu.VMEM((2, CHUNK), jnp.int32),
            pltpu.VMEM((2, CHUNK), jnp.int32),
            pltpu.VMEM((2, CHUNK), jnp.float32),
            pltpu.VMEM((2, CHUNK), jnp.float32),
            pltpu.VMEM((2, CHUNK), jnp.float32),
            pltpu.SemaphoreType.DMA,
            pltpu.SemaphoreType.DMA,
            pltpu.SemaphoreType.DMA,
            pltpu.SemaphoreType.DMA,
        ],
    )(_body)
    return kern(im1f, im2f, cf, m1f, m2f)


def kernel(im1, im2, C, M1, M2):
    outf = _run(
        im1.reshape(N * CH * HW),
        im2.reshape(N * CH * HW),
        C.reshape(N * 2 * HW),
        M1.reshape(N * CH * HW),
        M2.reshape(N * CH * HW),
    )
    return outf.reshape(N, CH, D, D)


# speedup vs baseline: 2.8262x; 1.3758x over previous
import functools

import jax
import jax.numpy as jnp
from jax import lax
from jax.experimental import pallas as pl
from jax.experimental.pallas import tpu as pltpu
from jax.experimental.pallas import tpu_sc as plsc

D = 224
HW = D * D
N = 64
CH = 3
NW = 32
SAMPLES_PER_W = N // NW
NTASK = SAMPLES_PER_W * CH
CHUNK_ROWS = 8
NCHUNKS = D // CHUNK_ROWS
VECS_PER_ROW = D // 16


def _body(im1_hbm, im2_hbm, c_hbm, m1_hbm, m2_hbm, out_hbm,
          g1v, g2v, c0v, c1v, m1v, m2v, ov, sin0, sin1, sout0, sout1):
    cid = lax.axis_index("c")
    sid = lax.axis_index("s")
    wid = sid * 2 + cid
    tbase = lax.iota(jnp.int32, 16) * D
    sin = (sin0, sin1)
    sout = (sout0, sout1)

    def in_copies(slot, n, ch, r0):
        return (
            pltpu.make_async_copy(c_hbm.at[n, 0, pl.ds(r0, CHUNK_ROWS), :],
                                  c0v.at[slot], sin[slot]),
            pltpu.make_async_copy(c_hbm.at[n, 1, pl.ds(r0, CHUNK_ROWS), :],
                                  c1v.at[slot], sin[slot]),
            pltpu.make_async_copy(m1_hbm.at[n, ch, pl.ds(r0, CHUNK_ROWS), :],
                                  m1v.at[slot], sin[slot]),
            pltpu.make_async_copy(m2_hbm.at[n, ch, pl.ds(r0, CHUNK_ROWS), :],
                                  m2v.at[slot], sin[slot]),
        )

    def out_copy(slot, n, ch, r0):
        return pltpu.make_async_copy(ov.at[slot],
                                     out_hbm.at[n, ch, pl.ds(r0, CHUNK_ROWS), :],
                                     sout[slot])

    def task_body(t, _):
        n = wid * SAMPLES_PER_W + t // CH
        ch = t % CH
        imbase = (n * CH + ch) * HW
        pltpu.sync_copy(im1_hbm.at[pl.ds(imbase, HW)], g1v)
        pltpu.sync_copy(im2_hbm.at[pl.ds(imbase, HW)], g2v)

        for c in in_copies(0, n, ch, 0):
            c.start()

        def compute_chunk(g, slot):
            x0 = g * CHUNK_ROWS

            def row_body(xr, _):
                x = x0 + xr
                for j in range(VECS_PER_ROW):
                    yoff = j * 16
                    c0 = c0v[slot, xr, pl.ds(yoff, 16)]
                    c1 = c1v[slot, xr, pl.ds(yoff, 16)]
                    dd = c0 + c1 * D
                    tv = tbase + (j * 16 * D + x)
                    i1 = jnp.minimum(tv + dd, HW - 1)
                    i2 = jnp.maximum(tv - dd, 0)
                    g1 = plsc.load_gather(g1v, [i1])
                    g2 = plsc.load_gather(g2v, [i2])
                    ov[slot, xr, pl.ds(yoff, 16)] = (
                        g1 * m1v[slot, xr, pl.ds(yoff, 16)]
                        + g2 * m2v[slot, xr, pl.ds(yoff, 16)])
                return 0

            lax.fori_loop(0, CHUNK_ROWS, row_body, 0)

        def pair_body(i, _):
            for sub in range(2):
                g = i * 2 + sub
                slot = sub
                nxt = g + 1

                @pl.when(nxt < NCHUNKS)
                def _():
                    for c in in_copies(1 - slot, n, ch, nxt * CHUNK_ROWS):
                        c.start()

                for c in in_copies(slot, n, ch, g * CHUNK_ROWS):
                    c.wait()

                @pl.when(g >= 2)
                def _():
                    out_copy(slot, n, ch, g * CHUNK_ROWS).wait()

                compute_chunk(g, slot)
                out_copy(slot, n, ch, g * CHUNK_ROWS).start()
            return 0

        lax.fori_loop(0, NCHUNKS // 2, pair_body, 0)
        out_copy(0, n, ch, (NCHUNKS - 2) * CHUNK_ROWS).wait()
        out_copy(1, n, ch, (NCHUNKS - 1) * CHUNK_ROWS).wait()
        return 0

    lax.fori_loop(0, NTASK, task_body, 0)


@jax.jit
def _run(im1f, im2f, c, m1, m2):
    mesh = plsc.VectorSubcoreMesh(core_axis_name="c", subcore_axis_name="s")
    kern = functools.partial(
        pl.kernel,
        mesh=mesh,
        compiler_params=pltpu.CompilerParams(needs_layout_passes=False),
        out_type=jax.ShapeDtypeStruct((N, CH, D, D), jnp.float32),
        scratch_types=[
            pltpu.VMEM((HW,), jnp.float32),
            pltpu.VMEM((HW,), jnp.float32),
            pltpu.VMEM((2, CHUNK_ROWS, D), jnp.int32),
            pltpu.VMEM((2, CHUNK_ROWS, D), jnp.int32),
            pltpu.VMEM((2, CHUNK_ROWS, D), jnp.float32),
            pltpu.VMEM((2, CHUNK_ROWS, D), jnp.float32),
            pltpu.VMEM((2, CHUNK_ROWS, D), jnp.float32),
            pltpu.SemaphoreType.DMA,
            pltpu.SemaphoreType.DMA,
            pltpu.SemaphoreType.DMA,
            pltpu.SemaphoreType.DMA,
        ],
    )(_body)
    return kern(im1f, im2f, c, m1, m2)


def kernel(im1, im2, C, M1, M2):
    return _run(
        im1.reshape(N * CH * HW),
        im2.reshape(N * CH * HW),
        C, M1, M2,
    )


# speedup vs baseline: 4.6147x; 1.6328x over previous
import functools

import jax
import jax.numpy as jnp
from jax import lax
from jax.experimental import pallas as pl
from jax.experimental.pallas import tpu as pltpu
from jax.experimental.pallas import tpu_sc as plsc

D = 224
HW = D * D
N = 64
CH = 3
NW = 32
SAMPLES_PER_W = N // NW
NTASK = SAMPLES_PER_W * CH
CHUNK_ROWS = 8
NCHUNKS = D // CHUNK_ROWS
VECS_PER_ROW = D // 16


def _body(im1_hbm, im2_hbm, c_hbm, m1_hbm, m2_hbm, out_hbm,
          g1v, g2v, c0v, c1v, m1v, m2v, ov, sin0, sin1, sout0, sout1):
    cid = lax.axis_index("c")
    sid = lax.axis_index("s")
    wid = sid * 2 + cid
    tbase = lax.iota(jnp.int32, 16) * D
    sin = (sin0, sin1)
    sout = (sout0, sout1)

    def in_copies(slot, n, ch, r0):
        return (
            pltpu.make_async_copy(c_hbm.at[n, 0, pl.ds(r0, CHUNK_ROWS), :],
                                  c0v.at[slot], sin[slot]),
            pltpu.make_async_copy(c_hbm.at[n, 1, pl.ds(r0, CHUNK_ROWS), :],
                                  c1v.at[slot], sin[slot]),
            pltpu.make_async_copy(m1_hbm.at[n, ch, pl.ds(r0, CHUNK_ROWS), :],
                                  m1v.at[slot], sin[slot]),
            pltpu.make_async_copy(m2_hbm.at[n, ch, pl.ds(r0, CHUNK_ROWS), :],
                                  m2v.at[slot], sin[slot]),
        )

    def out_copy(slot, n, ch, r0):
        return pltpu.make_async_copy(ov.at[slot],
                                     out_hbm.at[n, ch, pl.ds(r0, CHUNK_ROWS), :],
                                     sout[slot])

    def task_body(t, _):
        n = wid * SAMPLES_PER_W + t // CH
        ch = t % CH
        imbase = (n * CH + ch) * HW
        pltpu.sync_copy(im1_hbm.at[pl.ds(imbase, HW)], g1v)
        pltpu.sync_copy(im2_hbm.at[pl.ds(imbase, HW)], g2v)

        for c in in_copies(0, n, ch, 0):
            c.start()

        def compute_chunk(g, slot):
            x0 = g * CHUNK_ROWS

            @plsc.parallel_loop(0, CHUNK_ROWS)
            def row_body(xr):
                x = x0 + xr

                @plsc.parallel_loop(0, VECS_PER_ROW, 1, unroll=VECS_PER_ROW)
                def group_body(j):
                    yoff = j * 16
                    c0 = c0v[slot, xr, pl.ds(yoff, 16)]
                    c1 = c1v[slot, xr, pl.ds(yoff, 16)]
                    dd = c0 + c1 * D
                    tv = tbase + (j * (16 * D) + x)
                    i1 = jnp.minimum(tv + dd, HW - 1)
                    i2 = jnp.maximum(tv - dd, 0)
                    g1 = plsc.load_gather(g1v, [i1])
                    g2 = plsc.load_gather(g2v, [i2])
                    ov[slot, xr, pl.ds(yoff, 16)] = (
                        g1 * m1v[slot, xr, pl.ds(yoff, 16)]
                        + g2 * m2v[slot, xr, pl.ds(yoff, 16)])

        def pair_body(i, _):
            for sub in range(2):
                g = i * 2 + sub
                slot = sub
                nxt = g + 1

                @pl.when(nxt < NCHUNKS)
                def _():
                    for c in in_copies(1 - slot, n, ch, nxt * CHUNK_ROWS):
                        c.start()

                for c in in_copies(slot, n, ch, g * CHUNK_ROWS):
                    c.wait()

                @pl.when(g >= 2)
                def _():
                    out_copy(slot, n, ch, g * CHUNK_ROWS).wait()

                compute_chunk(g, slot)
                out_copy(slot, n, ch, g * CHUNK_ROWS).start()
            return 0

        lax.fori_loop(0, NCHUNKS // 2, pair_body, 0)
        out_copy(0, n, ch, (NCHUNKS - 2) * CHUNK_ROWS).wait()
        out_copy(1, n, ch, (NCHUNKS - 1) * CHUNK_ROWS).wait()
        return 0

    lax.fori_loop(0, NTASK, task_body, 0)


@jax.jit
def _run(im1f, im2f, c, m1, m2):
    mesh = plsc.VectorSubcoreMesh(core_axis_name="c", subcore_axis_name="s")
    kern = functools.partial(
        pl.kernel,
        mesh=mesh,
        compiler_params=pltpu.CompilerParams(needs_layout_passes=False),
        out_type=jax.ShapeDtypeStruct((N, CH, D, D), jnp.float32),
        scratch_types=[
            pltpu.VMEM((HW,), jnp.float32),
            pltpu.VMEM((HW,), jnp.float32),
            pltpu.VMEM((2, CHUNK_ROWS, D), jnp.int32),
            pltpu.VMEM((2, CHUNK_ROWS, D), jnp.int32),
            pltpu.VMEM((2, CHUNK_ROWS, D), jnp.float32),
            pltpu.VMEM((2, CHUNK_ROWS, D), jnp.float32),
            pltpu.VMEM((2, CHUNK_ROWS, D), jnp.float32),
            pltpu.SemaphoreType.DMA,
            pltpu.SemaphoreType.DMA,
            pltpu.SemaphoreType.DMA,
            pltpu.SemaphoreType.DMA,
        ],
    )(_body)
    return kern(im1f, im2f, c, m1, m2)


def kernel(im1, im2, C, M1, M2):
    return _run(
        im1.reshape(N * CH * HW),
        im2.reshape(N * CH * HW),
        C, M1, M2,
    )


# speedup vs baseline: 5.6035x; 1.2143x over previous
import functools

import jax
import jax.numpy as jnp
from jax import lax
from jax.experimental import pallas as pl
from jax.experimental.pallas import tpu as pltpu
from jax.experimental.pallas import tpu_sc as plsc

D = 224
HW = D * D
N = 64
CH = 3
NW = 32
SAMPLES_PER_W = N // NW
NTASK = SAMPLES_PER_W * CH
CHUNK_ROWS = 8
NCHUNKS = D // CHUNK_ROWS
VECS_PER_ROW = D // 16


NSLAB = D // 8
SPAIRS = NSLAB // 2


def _body(im1_hbm, im2_hbm, c_hbm, m1_hbm, m2_hbm, out_hbm,
          g1v, g2v, c0v, c1v, m1v, m2v, ov, stg, sin0, sin1, sout0, sout1,
          sstg0, sstg1):
    cid = lax.axis_index("c")
    sid = lax.axis_index("s")
    wid = sid * 2 + cid
    tbase = lax.iota(jnp.int32, 16) * D
    sin = (sin0, sin1)
    sout = (sout0, sout1)
    sstg = (sstg0, sstg1)

    def stage_image(src5, n, ch, dstflat):
        def pair_copy(p, sbuf):
            return pltpu.make_async_copy(
                src5.at[n, ch, pl.ds(p * 2, 2), :, :], stg.at[sbuf],
                sstg[sbuf])

        pair_copy(0, 0).start()

        def stage_pair(i, _):
            for sub in range(2):
                p = i * 2 + sub
                sbuf = sub

                @pl.when(p + 1 < SPAIRS)
                def _():
                    pair_copy(p + 1, 1 - sbuf).start()

                pair_copy(p, sbuf).wait()
                for sg in range(2):
                    rowbase = (p * 2 + sg) * 8

                    @plsc.parallel_loop(0, 8)
                    def _(rr):
                        flatbase = (rowbase + rr) * D

                        @plsc.parallel_loop(0, VECS_PER_ROW, 1,
                                            unroll=VECS_PER_ROW)
                        def _(j):
                            dstflat[pl.ds(flatbase + j * 16, 16)] = (
                                stg[sbuf, sg, rr, pl.ds(j * 16, 16)])
            return 0

        lax.fori_loop(0, SPAIRS // 2, stage_pair, 0)

    def in_copies(slot, n, ch, r0):
        return (
            pltpu.make_async_copy(c_hbm.at[n, 0, pl.ds(r0, CHUNK_ROWS), :],
                                  c0v.at[slot], sin[slot]),
            pltpu.make_async_copy(c_hbm.at[n, 1, pl.ds(r0, CHUNK_ROWS), :],
                                  c1v.at[slot], sin[slot]),
            pltpu.make_async_copy(m1_hbm.at[n, ch, pl.ds(r0, CHUNK_ROWS), :],
                                  m1v.at[slot], sin[slot]),
            pltpu.make_async_copy(m2_hbm.at[n, ch, pl.ds(r0, CHUNK_ROWS), :],
                                  m2v.at[slot], sin[slot]),
        )

    def out_copy(slot, n, ch, r0):
        return pltpu.make_async_copy(ov.at[slot],
                                     out_hbm.at[n, ch, pl.ds(r0, CHUNK_ROWS), :],
                                     sout[slot])

    def task_body(t, _):
        n = wid * SAMPLES_PER_W + t // CH
        ch = t % CH
        stage_image(im1_hbm, n, ch, g1v)
        stage_image(im2_hbm, n, ch, g2v)

        for c in in_copies(0, n, ch, 0):
            c.start()

        def compute_chunk(g, slot):
            x0 = g * CHUNK_ROWS

            @plsc.parallel_loop(0, CHUNK_ROWS)
            def row_body(xr):
                x = x0 + xr

                @plsc.parallel_loop(0, VECS_PER_ROW, 1, unroll=VECS_PER_ROW)
                def group_body(j):
                    yoff = j * 16
                    c0 = c0v[slot, xr, pl.ds(yoff, 16)]
                    c1 = c1v[slot, xr, pl.ds(yoff, 16)]
                    dd = c0 + c1 * D
                    tv = tbase + (j * (16 * D) + x)
                    i1 = jnp.minimum(tv + dd, HW - 1)
                    i2 = jnp.maximum(tv - dd, 0)
                    g1 = plsc.load_gather(g1v, [i1])
                    g2 = plsc.load_gather(g2v, [i2])
                    ov[slot, xr, pl.ds(yoff, 16)] = (
                        g1 * m1v[slot, xr, pl.ds(yoff, 16)]
                        + g2 * m2v[slot, xr, pl.ds(yoff, 16)])

        def pair_body(i, _):
            for sub in range(2):
                g = i * 2 + sub
                slot = sub
                nxt = g + 1

                @pl.when(nxt < NCHUNKS)
                def _():
                    for c in in_copies(1 - slot, n, ch, nxt * CHUNK_ROWS):
                        c.start()

                for c in in_copies(slot, n, ch, g * CHUNK_ROWS):
                    c.wait()

                @pl.when(g >= 2)
                def _():
                    out_copy(slot, n, ch, g * CHUNK_ROWS).wait()

                compute_chunk(g, slot)
                out_copy(slot, n, ch, g * CHUNK_ROWS).start()
            return 0

        lax.fori_loop(0, NCHUNKS // 2, pair_body, 0)
        out_copy(0, n, ch, (NCHUNKS - 2) * CHUNK_ROWS).wait()
        out_copy(1, n, ch, (NCHUNKS - 1) * CHUNK_ROWS).wait()
        return 0

    lax.fori_loop(0, NTASK, task_body, 0)


@jax.jit
def _run(im1f, im2f, c, m1, m2):
    mesh = plsc.VectorSubcoreMesh(core_axis_name="c", subcore_axis_name="s")
    kern = functools.partial(
        pl.kernel,
        mesh=mesh,
        compiler_params=pltpu.CompilerParams(needs_layout_passes=False),
        out_type=jax.ShapeDtypeStruct((N, CH, D, D), jnp.float32),
        scratch_types=[
            pltpu.VMEM((HW,), jnp.float32),
            pltpu.VMEM((HW,), jnp.float32),
            pltpu.VMEM((2, CHUNK_ROWS, D), jnp.int32),
            pltpu.VMEM((2, CHUNK_ROWS, D), jnp.int32),
            pltpu.VMEM((2, CHUNK_ROWS, D), jnp.float32),
            pltpu.VMEM((2, CHUNK_ROWS, D), jnp.float32),
            pltpu.VMEM((2, CHUNK_ROWS, D), jnp.float32),
            pltpu.VMEM((2, 2, 8, D), jnp.float32),
            pltpu.SemaphoreType.DMA,
            pltpu.SemaphoreType.DMA,
            pltpu.SemaphoreType.DMA,
            pltpu.SemaphoreType.DMA,
            pltpu.SemaphoreType.DMA,
            pltpu.SemaphoreType.DMA,
        ],
    )(_body)
    return kern(im1f, im2f, c, m1, m2)


def kernel(im1, im2, C, M1, M2):
    return _run(
        im1.reshape(N, CH, NSLAB, 8, D),
        im2.reshape(N, CH, NSLAB, 8, D),
        C, M1, M2,
    )


# speedup vs baseline: 5.6370x; 1.0060x over previous
import functools

import jax
import jax.numpy as jnp
from jax import lax
from jax.experimental import pallas as pl
from jax.experimental.pallas import tpu as pltpu
from jax.experimental.pallas import tpu_sc as plsc

D = 224
HW = D * D
N = 64
CH = 3
NW = 32
SAMPLES_PER_W = N // NW
NTASK = SAMPLES_PER_W * CH
CHUNK_ROWS = 8
NCHUNKS = D // CHUNK_ROWS
VECS_PER_ROW = D // 16


NSLAB = D // 8
SPAIRS = NSLAB // 2


def _body(im1_hbm, im2_hbm, c_hbm, m1_hbm, m2_hbm, out_hbm,
          g1v, g2v, c0v, c1v, m1v, m2v, ov, stg, sin0, sin1, sout0, sout1,
          sstg0, sstg1):
    cid = lax.axis_index("c")
    sid = lax.axis_index("s")
    wid = sid * 2 + cid
    tbase = lax.iota(jnp.int32, 16) * D
    sin = (sin0, sin1)
    sout = (sout0, sout1)
    sstg = (sstg0, sstg1)

    def stage_image(src5, n, ch, dstflat):
        def pair_copy(p, sbuf):
            return pltpu.make_async_copy(
                src5.at[n, ch, pl.ds(p * 2, 2), :, :], stg.at[sbuf],
                sstg[sbuf])

        pair_copy(0, 0).start()

        def stage_pair(i, _):
            for sub in range(2):
                p = i * 2 + sub
                sbuf = sub

                @pl.when(p + 1 < SPAIRS)
                def _():
                    pair_copy(p + 1, 1 - sbuf).start()

                pair_copy(p, sbuf).wait()
                for sg in range(2):
                    rowbase = (p * 2 + sg) * 8

                    @plsc.parallel_loop(0, 8, 1, unroll=4)
                    def _(rr):
                        flatbase = (rowbase + rr) * D

                        @plsc.parallel_loop(0, VECS_PER_ROW, 1,
                                            unroll=VECS_PER_ROW)
                        def _(j):
                            dstflat[pl.ds(flatbase + j * 16, 16)] = (
                                stg[sbuf, sg, rr, pl.ds(j * 16, 16)])
            return 0

        lax.fori_loop(0, SPAIRS // 2, stage_pair, 0)

    def in_copies(slot, n, ch, r0):
        return (
            pltpu.make_async_copy(c_hbm.at[n, 0, pl.ds(r0, CHUNK_ROWS), :],
                                  c0v.at[slot], sin[slot]),
            pltpu.make_async_copy(c_hbm.at[n, 1, pl.ds(r0, CHUNK_ROWS), :],
                                  c1v.at[slot], sin[slot]),
            pltpu.make_async_copy(m1_hbm.at[n, ch, pl.ds(r0, CHUNK_ROWS), :],
                                  m1v.at[slot], sin[slot]),
            pltpu.make_async_copy(m2_hbm.at[n, ch, pl.ds(r0, CHUNK_ROWS), :],
                                  m2v.at[slot], sin[slot]),
        )

    def out_copy(slot, n, ch, r0):
        return pltpu.make_async_copy(ov.at[slot],
                                     out_hbm.at[n, ch, pl.ds(r0, CHUNK_ROWS), :],
                                     sout[slot])

    def task_body(t, _):
        n = wid * SAMPLES_PER_W + t // CH
        ch = t % CH
        for c in in_copies(0, n, ch, 0):
            c.start()
        stage_image(im1_hbm, n, ch, g1v)
        stage_image(im2_hbm, n, ch, g2v)

        def compute_chunk(g, slot):
            x0 = g * CHUNK_ROWS

            @plsc.parallel_loop(0, CHUNK_ROWS)
            def row_body(xr):
                x = x0 + xr

                @plsc.parallel_loop(0, VECS_PER_ROW, 1, unroll=VECS_PER_ROW)
                def group_body(j):
                    yoff = j * 16
                    c0 = c0v[slot, xr, pl.ds(yoff, 16)]
                    c1 = c1v[slot, xr, pl.ds(yoff, 16)]
                    dd = c0 + c1 * D
                    tv = tbase + (j * (16 * D) + x)
                    i1 = jnp.minimum(tv + dd, HW - 1)
                    i2 = jnp.maximum(tv - dd, 0)
                    g1 = plsc.load_gather(g1v, [i1])
                    g2 = plsc.load_gather(g2v, [i2])
                    ov[slot, xr, pl.ds(yoff, 16)] = (
                        g1 * m1v[slot, xr, pl.ds(yoff, 16)]
                        + g2 * m2v[slot, xr, pl.ds(yoff, 16)])

        def pair_body(i, _):
            for sub in range(2):
                g = i * 2 + sub
                slot = sub
                nxt = g + 1

                @pl.when(nxt < NCHUNKS)
                def _():
                    for c in in_copies(1 - slot, n, ch, nxt * CHUNK_ROWS):
                        c.start()

                for c in in_copies(slot, n, ch, g * CHUNK_ROWS):
                    c.wait()

                @pl.when(g >= 2)
                def _():
                    out_copy(slot, n, ch, g * CHUNK_ROWS).wait()

                compute_chunk(g, slot)
                out_copy(slot, n, ch, g * CHUNK_ROWS).start()
            return 0

        lax.fori_loop(0, NCHUNKS // 2, pair_body, 0)
        out_copy(0, n, ch, (NCHUNKS - 2) * CHUNK_ROWS).wait()
        out_copy(1, n, ch, (NCHUNKS - 1) * CHUNK_ROWS).wait()
        return 0

    lax.fori_loop(0, NTASK, task_body, 0)


@jax.jit
def _run(im1f, im2f, c, m1, m2):
    mesh = plsc.VectorSubcoreMesh(core_axis_name="c", subcore_axis_name="s")
    kern = functools.partial(
        pl.kernel,
        mesh=mesh,
        compiler_params=pltpu.CompilerParams(needs_layout_passes=False),
        out_type=jax.ShapeDtypeStruct((N, CH, D, D), jnp.float32),
        scratch_types=[
            pltpu.VMEM((HW,), jnp.float32),
            pltpu.VMEM((HW,), jnp.float32),
            pltpu.VMEM((2, CHUNK_ROWS, D), jnp.int32),
            pltpu.VMEM((2, CHUNK_ROWS, D), jnp.int32),
            pltpu.VMEM((2, CHUNK_ROWS, D), jnp.float32),
            pltpu.VMEM((2, CHUNK_ROWS, D), jnp.float32),
            pltpu.VMEM((2, CHUNK_ROWS, D), jnp.float32),
            pltpu.VMEM((2, 2, 8, D), jnp.float32),
            pltpu.SemaphoreType.DMA,
            pltpu.SemaphoreType.DMA,
            pltpu.SemaphoreType.DMA,
            pltpu.SemaphoreType.DMA,
            pltpu.SemaphoreType.DMA,
            pltpu.SemaphoreType.DMA,
        ],
    )(_body)
    return kern(im1f, im2f, c, m1, m2)


def kernel(im1, im2, C, M1, M2):
    return _run(
        im1.reshape(N, CH, NSLAB, 8, D),
        im2.reshape(N, CH, NSLAB, 8, D),
        C, M1, M2,
    )
